# Initial kernel scaffold; baseline (speedup 1.0000x reference)
import jax
import jax.numpy as jnp
from jax.experimental import pallas as pl


def kernel(params, h, e, edge_index):
    raise NotImplementedError("write your pallas kernel here")


# speedup vs baseline: 2.5855x; 2.5855x over previous
import functools

import jax
import jax.numpy as jnp
import numpy as np
from jax import lax
from jax.experimental import pallas as pl
from jax.experimental.pallas import tpu as pltpu
from jax.experimental.pallas import tpu_sc as plsc

N_NODES = 10000
N_EDGES = 320000
HID = 128
HEADS = 8
HDIM = 16
NUM_ATOM = 100
NUM_BOND = 10

NC, NS = 2, 16
NPAD = 10240
NH = NPAD // 2
NZH = NH // 16
NACC = NH + NZH
CH1 = 64
CH2 = 64
INV_SQRT_D = float(1.0 / np.sqrt(HDIM))

_sc_params = pltpu.CompilerParams(needs_layout_passes=False)


def _iota16():
    return jnp.arange(16, dtype=jnp.int32)


def _zero_rows_loop(buf, nrows, ncols):
    z16 = jnp.zeros((16,), jnp.float32)

    def body(r, _):
        for j in range(ncols // 16):
            buf[r, pl.ds(j * 16, 16)] = z16
        return 0

    lax.fori_loop(0, nrows, body, 0)


def _zero_shared(zrow_v, acc_sh, nrows_total, sid):
    rows_per = nrows_total // NS
    for j in range(rows_per // 16):
        off = pl.multiple_of(sid * rows_per + j * 16, 16)
        pltpu.sync_copy(zrow_v, acc_sh.at[pl.ds(off, 16)])


def _zero_shared_units(zrow_v, acc, sid):
    units = NACC // 8
    myn = jnp.where(sid < 8, (units + 15) // 16, units // 16)
    base = jnp.where(sid < 8, sid * ((units + 15) // 16),
                     8 * ((units + 15) // 16) + (sid - 8) * (units // 16))

    def body(u, _):
        @pl.when(u < myn)
        def _():
            off = pl.multiple_of((base + u) * 8, 8)
            pltpu.sync_copy(zrow_v.at[pl.ds(0, 8)], acc.at[pl.ds(off, 8)])
        return 0

    lax.fori_loop(0, (units + 15) // 16, body, 0)


def _dump_shared_units(acc, parts_hbm, cid, sid):
    units = NACC // 8
    myn = jnp.where(sid < 8, (units + 15) // 16, units // 16)
    base = jnp.where(sid < 8, sid * ((units + 15) // 16),
                     8 * ((units + 15) // 16) + (sid - 8) * (units // 16))

    def body(u, _):
        @pl.when(u < myn)
        def _():
            off = pl.multiple_of((base + u) * 8, 8)
            pltpu.sync_copy(acc.at[pl.ds(off, 8)],
                            parts_hbm.at[cid, pl.ds(off, 8)])
        return 0

    lax.fori_loop(0, (units + 15) // 16, body, 0)


def _zero_shared_z(zrow_v, acc_z, sid):
    @pl.when(sid < 8)
    def _():
        for j in range(5):
            off = pl.multiple_of(sid * 40 + j * 8, 8)
            pltpu.sync_copy(zrow_v.at[pl.ds(0, 8)], acc_z.at[pl.ds(off, 8)])


def _dump_shared_z(acc_z, zparts_hbm, cid, sid):
    @pl.when(sid < 8)
    def _():
        off = pl.multiple_of(sid * 40, 8)
        pltpu.sync_copy(acc_z.at[pl.ds(off, 40)],
                        zparts_hbm.at[cid, pl.ds(off, 40)])


def _dump_shared(acc_sh, parts_hbm, nrows_total, cid, sid):
    rows_per = nrows_total // NS
    off = pl.multiple_of(sid * rows_per, 8)
    pltpu.sync_copy(acc_sh.at[pl.ds(off, rows_per)],
                    parts_hbm.at[cid, pl.ds(off, rows_per)])


def _edge_rows(g, loadk, loadq, loade, loadv,
               d16, dlc16, lo, sv_v, zv_v, sc_v=None):
    iota = _iota16()
    for l in range(16):
        dstv = d16[l]
        dlc = dlc16[l]
        inr = (dstv >= lo) & (dstv < lo + NH)
        svecs = []
        for hh in range(HEADS):
            p = loadk(l, hh) * loadq(l, hh) * loade(l, hh)
            if sc_v is not None:
                sc_v[g * 16 + l, pl.ds(hh * HDIM, 16)] = p
            tot = jnp.clip(jnp.sum(p), -5.0, 5.0)
            sv = jnp.where(inr, jnp.exp(jnp.full((16,), tot, jnp.float32)),
                           jnp.zeros((16,), jnp.float32))
            svecs.append(sv)
            if sv_v is not None:
                sv_v[g * 16 + l, pl.ds(hh * HDIM, 16)] = loadv(l, hh) * sv
        if zv_v is None:
            continue
        base = (dlc & 15) * 8
        for j in range(HID // 16):
            zvj = jnp.zeros((16,), jnp.float32)
            for hh in range(HEADS):
                zvj = jnp.where(iota == base + hh - j * 16, svecs[hh], zvj)
            zv_v[g * 16 + l, pl.ds(j * 16, 16)] = zvj


def _make_sc1():
    mesh = plsc.VectorSubcoreMesh(core_axis_name="c", subcore_axis_name="s")
    TOT = N_EDGES // CH1

    @functools.partial(
        pl.kernel, mesh=mesh, compiler_params=_sc_params,
        out_type=(jax.ShapeDtypeStruct((N_EDGES, HID), jnp.float32),
                  jax.ShapeDtypeStruct((NC, NH, HID), jnp.float32)),
        scratch_types=(
            [pltpu.VMEM((NUM_ATOM, HID), jnp.float32)] * 3 +
            [pltpu.VMEM((NUM_BOND, HID), jnp.float32)] +
            [pltpu.VMEM((CH1,), jnp.int32)] * 10 +
            [pltpu.VMEM((CH1,), jnp.int32)] * 2 +
            [pltpu.VMEM((CH1, HID), jnp.float32)] * 4 +
            [pltpu.VMEM((16, HID), jnp.float32)] +
            [pltpu.VMEM_SHARED((NH, HID), jnp.float32)] +
            [pltpu.SemaphoreType.DMA] * 6
        ),
    )
    def sc1(mk_hbm, mq_hbm, mv_hbm, me_hbm, aidx_hbm, bidx_hbm, eidx_hbm,
            dst_hbm, dstl_hbm, erow_hbm,
            score_hbm, parts_hbm,
            mk_v, mq_v, mv_v, me_v,
            a0, b0, e0, d0, dl0, a1, b1, e1, d1, dl1,
            sci0, sci1, sc0, sc1b, sv0, sv1,
            zrow_v, acc_wv,
            isem0, isem1, ssem0, ssem1, vsem0, vsem1):
        cid = lax.axis_index("c")
        sid = lax.axis_index("s")
        lo = cid * NH
        as_ = (a0, a1)
        bs = (b0, b1)
        es = (e0, e1)
        ds_ = (d0, d1)
        dls = (dl0, dl1)
        scis = (sci0, sci1)
        scs = (sc0, sc1b)
        svs = (sv0, sv1)
        isems = (isem0, isem1)
        ssems = (ssem0, ssem1)
        vsems = (vsem0, vsem1)

        trips = (TOT - sid + NS - 1) // NS

        def chunk_of(i):
            return sid + NS * i

        def fire_in(b, i):
            eb = chunk_of(i) * CH1
            pltpu.async_copy(aidx_hbm.at[pl.ds(eb, CH1)], as_[b], isems[b])
            pltpu.async_copy(bidx_hbm.at[pl.ds(eb, CH1)], bs[b], isems[b])
            pltpu.async_copy(eidx_hbm.at[pl.ds(eb, CH1)], es[b], isems[b])
            pltpu.async_copy(dst_hbm.at[pl.ds(eb, CH1)], ds_[b], isems[b])
            pltpu.async_copy(dstl_hbm.at[cid, pl.ds(eb, CH1)], dls[b], isems[b])
            pltpu.async_copy(erow_hbm.at[pl.ds(eb, CH1)], scis[b], isems[b])

        def wait_in(b):
            pltpu.make_async_copy(aidx_hbm.at[pl.ds(0, CH1)], as_[b], isems[b]).wait()
            pltpu.make_async_copy(bidx_hbm.at[pl.ds(0, CH1)], bs[b], isems[b]).wait()
            pltpu.make_async_copy(eidx_hbm.at[pl.ds(0, CH1)], es[b], isems[b]).wait()
            pltpu.make_async_copy(dst_hbm.at[pl.ds(0, CH1)], ds_[b], isems[b]).wait()
            pltpu.make_async_copy(dstl_hbm.at[cid, pl.ds(0, CH1)], dls[b],
                                  isems[b]).wait()
            pltpu.make_async_copy(erow_hbm.at[pl.ds(0, CH1)], scis[b],
                                  isems[b]).wait()

        def wait_score(b):
            pltpu.make_async_copy(scs[b], score_hbm.at[scis[b]], ssems[b]).wait()

        def wait_out(b):
            pltpu.make_async_copy(svs[b], acc_wv.at[dls[b]], vsems[b]).wait()

        def fire_out(b):
            pltpu.async_copy(svs[b], acc_wv.at[dls[b]], vsems[b], add=True)

        pltpu.sync_copy(mk_hbm, mk_v)
        pltpu.sync_copy(mq_hbm, mq_v)
        pltpu.sync_copy(mv_hbm, mv_v)
        pltpu.sync_copy(me_hbm, me_v)
        z16 = jnp.zeros((16,), jnp.float32)
        z16i = jnp.zeros((16,), jnp.int32)
        for r in range(16):
            for j in range(HID // 16):
                zrow_v[r, pl.ds(j * 16, 16)] = z16
        for b in (0, 1):
            for j in range(CH1 // 16):
                dls[b][pl.ds(j * 16, 16)] = z16i
        for b in (0, 1):
            _zero_rows_loop(svs[b], CH1, HID)
        _zero_shared(zrow_v, acc_wv, NH, sid)
        plsc.subcore_barrier()
        fire_out(0)
        fire_out(1)

        fire_in(0, 0)
        fire_in(1, 1)

        def pair_body(pp, _):
            for b in (0, 1):
                i = 2 * pp + b

                @pl.when(i < trips)
                def _():
                    wait_in(b)
                    wait_out(b)

                    @pl.when((i >= 2) & (cid == 0))
                    def _():
                        wait_score(b)

                    def gbody(g, _):
                        a16 = as_[b][pl.ds(g * 16, 16)]
                        b16 = bs[b][pl.ds(g * 16, 16)]
                        e16 = es[b][pl.ds(g * 16, 16)]
                        d16 = ds_[b][pl.ds(g * 16, 16)]
                        dlc16 = dls[b][pl.ds(g * 16, 16)]
                        _edge_rows(
                            g,
                            lambda l, hh: mk_v[a16[l], pl.ds(hh * HDIM, 16)],
                            lambda l, hh: mq_v[b16[l], pl.ds(hh * HDIM, 16)],
                            lambda l, hh: me_v[e16[l], pl.ds(hh * HDIM, 16)],
                            lambda l, hh: mv_v[a16[l], pl.ds(hh * HDIM, 16)],
                            d16, dlc16, lo, svs[b], None, sc_v=scs[b])
                        return 0

                    lax.fori_loop(0, CH1 // 16, gbody, 0)

                    @pl.when(cid == 0)
                    def _():
                        pltpu.async_copy(scs[b], score_hbm.at[scis[b]],
                                         ssems[b])
                    fire_out(b)

                    @pl.when(i + 2 < trips)
                    def _():
                        fire_in(b, i + 2)
            return 0

        lax.fori_loop(0, (trips + 1) // 2, pair_body, 0)
        wait_out(0)
        wait_out(1)

        @pl.when(cid == 0)
        def _():
            wait_score(0)
            wait_score(1)
        plsc.subcore_barrier()
        _dump_shared(acc_wv, parts_hbm, NH, cid, sid)

    return sc1


def _make_sc2():
    mesh = plsc.VectorSubcoreMesh(core_axis_name="c", subcore_axis_name="s")
    TOT = N_EDGES // CH2

    @functools.partial(
        pl.kernel, mesh=mesh, compiler_params=_sc_params,
        out_type=jax.ShapeDtypeStruct((NC, NH, HID), jnp.float32),
        scratch_types=(
            [pltpu.VMEM((CH2,), jnp.int32)] * 8 +
            [pltpu.VMEM((CH2, 2 * HID), jnp.float32)] * 2 +
            [pltpu.VMEM((CH2, HID), jnp.float32)] * 4 +
            [pltpu.VMEM((CH2, HID), jnp.float32)] * 2 +
            [pltpu.VMEM((16, HID), jnp.float32)] +
            [pltpu.VMEM_SHARED((NH, HID), jnp.float32)] +
            [pltpu.SemaphoreType.DMA] * 6
        ),
    )
    def sc2(kv_hbm, q_hbm, ee_hbm, src_hbm, dst_hbm, dstl_hbm,
            erow_hbm, parts_hbm,
            s0, d0, dl0, ei0, s1, d1, dl1, ei1,
            kvr0, kvr1, qr0, qr1, er0, er1, sv0, sv1,
            zrow_v, acc_wv,
            isem0, isem1, gsem0, gsem1, vsem0, vsem1):
        cid = lax.axis_index("c")
        sid = lax.axis_index("s")
        lo = cid * NH
        srcs = (s0, s1)
        ds_ = (d0, d1)
        dls = (dl0, dl1)
        eeis = (ei0, ei1)
        kvrs = (kvr0, kvr1)
        qrs = (qr0, qr1)
        ers = (er0, er1)
        svs = (sv0, sv1)
        isems = (isem0, isem1)
        gsems = (gsem0, gsem1)
        vsems = (vsem0, vsem1)

        trips = (TOT - sid + NS - 1) // NS

        def chunk_of(i):
            return sid + NS * i

        def fire_idx(b, i):
            eb = chunk_of(i) * CH2
            pltpu.async_copy(src_hbm.at[pl.ds(eb, CH2)], srcs[b], isems[b])
            pltpu.async_copy(dst_hbm.at[pl.ds(eb, CH2)], ds_[b], isems[b])
            pltpu.async_copy(dstl_hbm.at[cid, pl.ds(eb, CH2)], dls[b], isems[b])
            pltpu.async_copy(erow_hbm.at[pl.ds(eb, CH2)], eeis[b], isems[b])

        def wait_idx(b):
            pltpu.make_async_copy(src_hbm.at[pl.ds(0, CH2)], srcs[b], isems[b]).wait()
            pltpu.make_async_copy(dst_hbm.at[pl.ds(0, CH2)], ds_[b], isems[b]).wait()
            pltpu.make_async_copy(dstl_hbm.at[cid, pl.ds(0, CH2)], dls[b],
                                  isems[b]).wait()
            pltpu.make_async_copy(erow_hbm.at[pl.ds(0, CH2)], eeis[b],
                                  isems[b]).wait()

        def fire_gather(b):
            pltpu.async_copy(kv_hbm.at[srcs[b]], kvrs[b], gsems[b])
            pltpu.async_copy(q_hbm.at[ds_[b]], qrs[b], gsems[b])
            pltpu.async_copy(ee_hbm.at[eeis[b]], ers[b], gsems[b])

        def wait_gather(b):
            pltpu.make_async_copy(kv_hbm.at[srcs[b]], kvrs[b], gsems[b]).wait()
            pltpu.make_async_copy(q_hbm.at[ds_[b]], qrs[b], gsems[b]).wait()
            pltpu.make_async_copy(ee_hbm.at[eeis[b]], ers[b], gsems[b]).wait()

        def wait_out(b):
            pltpu.make_async_copy(svs[b], acc_wv.at[dls[b]], vsems[b]).wait()

        def fire_out(b):
            pltpu.async_copy(svs[b], acc_wv.at[dls[b]], vsems[b], add=True)

        z16 = jnp.zeros((16,), jnp.float32)
        z16i = jnp.zeros((16,), jnp.int32)
        for r in range(16):
            for j in range(HID // 16):
                zrow_v[r, pl.ds(j * 16, 16)] = z16
        for b in (0, 1):
            for j in range(CH2 // 16):
                dls[b][pl.ds(j * 16, 16)] = z16i
        for b in (0, 1):
            _zero_rows_loop(svs[b], CH2, HID)
        _zero_shared(zrow_v, acc_wv, NH, sid)
        plsc.subcore_barrier()
        fire_out(0)
        fire_out(1)

        fire_idx(0, 0)
        wait_idx(0)
        fire_gather(0)
        fire_idx(1, 1)

        def pair_body(pp, _):
            for b in (0, 1):
                i = 2 * pp + b

                @pl.when(i < trips)
                def _():
                    wait_gather(b)
                    wait_out(b)

                    @pl.when(i + 1 < trips)
                    def _():
                        wait_idx(1 - b)
                        fire_gather(1 - b)

                    def gbody(g, _):
                        d16 = ds_[b][pl.ds(g * 16, 16)]
                        dlc16 = dls[b][pl.ds(g * 16, 16)]
                        _edge_rows(
                            g,
                            lambda l, hh: kvrs[b][g * 16 + l,
                                                  pl.ds(hh * HDIM, 16)],
                            lambda l, hh: qrs[b][g * 16 + l,
                                                 pl.ds(hh * HDIM, 16)],
                            lambda l, hh: ers[b][g * 16 + l,
                                                 pl.ds(hh * HDIM, 16)]
                            * INV_SQRT_D,
                            lambda l, hh: kvrs[b][g * 16 + l,
                                                  pl.ds(HID + hh * HDIM, 16)],
                            d16, dlc16, lo, svs[b], None)
                        return 0

                    lax.fori_loop(0, CH2 // 16, gbody, 0)
                    fire_out(b)

                    @pl.when(i + 2 < trips)
                    def _():
                        fire_idx(b, i + 2)
            return 0

        lax.fori_loop(0, (trips + 1) // 2, pair_body, 0)
        wait_out(0)
        wait_out(1)
        plsc.subcore_barrier()
        _dump_shared(acc_wv, parts_hbm, NH, cid, sid)

    return sc2


def _make_scz1():
    mesh = plsc.VectorSubcoreMesh(core_axis_name="c", subcore_axis_name="s")
    TOT = N_EDGES // CH1

    @functools.partial(
        pl.kernel, mesh=mesh, compiler_params=_sc_params,
        out_type=jax.ShapeDtypeStruct((NC, NZH, HID), jnp.float32),
        scratch_types=(
            [pltpu.VMEM((NUM_ATOM, HID), jnp.float32)] * 2 +
            [pltpu.VMEM((NUM_BOND, HID), jnp.float32)] +
            [pltpu.VMEM((CH1,), jnp.int32)] * 10 +
            [pltpu.VMEM((CH1, HID), jnp.float32)] * 2 +
            [pltpu.VMEM((16, HID), jnp.float32)] +
            [pltpu.VMEM_SHARED((NZH, HID), jnp.float32)] +
            [pltpu.SemaphoreType.DMA] * 4
        ),
    )
    def scz1(mk_hbm, mq_hbm, me_hbm, aidx_hbm, bidx_hbm, eidx_hbm,
             dst_hbm, dstzl_hbm, zparts_hbm,
             mk_v, mq_v, me_v,
             a0, b0, e0, d0, dl0, a1, b1, e1, d1, dl1,
             zv0, zv1, zrow_v, acc_z,
             isem0, isem1, vsem0, vsem1):
        cid = lax.axis_index("c")
        sid = lax.axis_index("s")
        lo = cid * NH
        as_ = (a0, a1)
        bs = (b0, b1)
        es = (e0, e1)
        ds_ = (d0, d1)
        dls = (dl0, dl1)
        zvs = (zv0, zv1)
        isems = (isem0, isem1)
        vsems = (vsem0, vsem1)

        trips = (TOT - sid + NS - 1) // NS

        def chunk_of(i):
            return sid + NS * i

        def fire_in(b, i):
            eb = chunk_of(i) * CH1
            pltpu.async_copy(aidx_hbm.at[pl.ds(eb, CH1)], as_[b], isems[b])
            pltpu.async_copy(bidx_hbm.at[pl.ds(eb, CH1)], bs[b], isems[b])
            pltpu.async_copy(eidx_hbm.at[pl.ds(eb, CH1)], es[b], isems[b])
            pltpu.async_copy(dst_hbm.at[pl.ds(eb, CH1)], ds_[b], isems[b])
            pltpu.async_copy(dstzl_hbm.at[cid, pl.ds(eb, CH1)], dls[b], isems[b])

        def wait_in(b):
            pltpu.make_async_copy(aidx_hbm.at[pl.ds(0, CH1)], as_[b], isems[b]).wait()
            pltpu.make_async_copy(bidx_hbm.at[pl.ds(0, CH1)], bs[b], isems[b]).wait()
            pltpu.make_async_copy(eidx_hbm.at[pl.ds(0, CH1)], es[b], isems[b]).wait()
            pltpu.make_async_copy(dst_hbm.at[pl.ds(0, CH1)], ds_[b], isems[b]).wait()
            pltpu.make_async_copy(dstzl_hbm.at[cid, pl.ds(0, CH1)], dls[b],
                                  isems[b]).wait()

        def wait_out(b):
            pltpu.make_async_copy(zvs[b], acc_z.at[dls[b]], vsems[b]).wait()

        def fire_out(b):
            pltpu.async_copy(zvs[b], acc_z.at[dls[b]], vsems[b], add=True)

        pltpu.sync_copy(mk_hbm, mk_v)
        pltpu.sync_copy(mq_hbm, mq_v)
        pltpu.sync_copy(me_hbm, me_v)
        z16 = jnp.zeros((16,), jnp.float32)
        z16i = jnp.zeros((16,), jnp.int32)
        for r in range(16):
            for j in range(HID // 16):
                zrow_v[r, pl.ds(j * 16, 16)] = z16
        for b in (0, 1):
            for j in range(CH1 // 16):
                dls[b][pl.ds(j * 16, 16)] = z16i
            _zero_rows_loop(zvs[b], CH1, HID)
        _zero_shared_z(zrow_v, acc_z, sid)
        plsc.subcore_barrier()
        fire_out(0)
        fire_out(1)

        fire_in(0, 0)
        fire_in(1, 1)

        def pair_body(pp, _):
            for b in (0, 1):
                i = 2 * pp + b

                @pl.when(i < trips)
                def _():
                    wait_in(b)
                    wait_out(b)

                    def gbody(g, _):
                        a16 = as_[b][pl.ds(g * 16, 16)]
                        b16 = bs[b][pl.ds(g * 16, 16)]
                        e16 = es[b][pl.ds(g * 16, 16)]
                        d16 = ds_[b][pl.ds(g * 16, 16)]
                        dlc16 = jnp.clip(d16 - lo, 0, NH - 1)
                        _edge_rows(
                            g,
                            lambda l, hh: mk_v[a16[l], pl.ds(hh * HDIM, 16)],
                            lambda l, hh: mq_v[b16[l], pl.ds(hh * HDIM, 16)],
                            lambda l, hh: me_v[e16[l], pl.ds(hh * HDIM, 16)],
                            None,
                            d16, dlc16, lo, None, zvs[b])
                        return 0

                    lax.fori_loop(0, CH1 // 16, gbody, 0)
                    fire_out(b)

                    @pl.when(i + 2 < trips)
                    def _():
                        fire_in(b, i + 2)
            return 0

        lax.fori_loop(0, (trips + 1) // 2, pair_body, 0)
        wait_out(0)
        wait_out(1)
        plsc.subcore_barrier()
        _dump_shared_z(acc_z, zparts_hbm, cid, sid)

    return scz1


def _make_scz2():
    mesh = plsc.VectorSubcoreMesh(core_axis_name="c", subcore_axis_name="s")
    TOT = N_EDGES // CH2

    @functools.partial(
        pl.kernel, mesh=mesh, compiler_params=_sc_params,
        out_type=jax.ShapeDtypeStruct((NC, NZH, HID), jnp.float32),
        scratch_types=(
            [pltpu.VMEM((CH2,), jnp.int32)] * 8 +
            [pltpu.VMEM((CH2, 2 * HID), jnp.float32)] * 2 +
            [pltpu.VMEM((CH2, HID), jnp.float32)] * 4 +
            [pltpu.VMEM((CH2, HID), jnp.float32)] * 2 +
            [pltpu.VMEM((16, HID), jnp.float32)] +
            [pltpu.VMEM_SHARED((NZH, HID), jnp.float32)] +
            [pltpu.SemaphoreType.DMA] * 6
        ),
    )
    def scz2(kv_hbm, q_hbm, ee_hbm, src_hbm, dst_hbm, dstzl_hbm, erow_hbm,
             zparts_hbm,
             s0, d0, dl0, ei0, s1, d1, dl1, ei1,
             kr0, kr1, qr0, qr1, er0, er1, zv0, zv1,
             zrow_v, acc_z,
             isem0, isem1, gsem0, gsem1, vsem0, vsem1):
        cid = lax.axis_index("c")
        sid = lax.axis_index("s")
        lo = cid * NH
        srcs = (s0, s1)
        ds_ = (d0, d1)
        dls = (dl0, dl1)
        eeis = (ei0, ei1)
        krs = (kr0, kr1)
        qrs = (qr0, qr1)
        ers = (er0, er1)
        zvs = (zv0, zv1)
        isems = (isem0, isem1)
        gsems = (gsem0, gsem1)
        vsems = (vsem0, vsem1)

        trips = (TOT - sid + NS - 1) // NS

        def chunk_of(i):
            return sid + NS * i

        def fire_idx(b, i):
            eb = chunk_of(i) * CH2
            pltpu.async_copy(src_hbm.at[pl.ds(eb, CH2)], srcs[b], isems[b])
            pltpu.async_copy(dst_hbm.at[pl.ds(eb, CH2)], ds_[b], isems[b])
            pltpu.async_copy(dstzl_hbm.at[cid, pl.ds(eb, CH2)], dls[b], isems[b])
            pltpu.async_copy(erow_hbm.at[pl.ds(eb, CH2)], eeis[b], isems[b])

        def wait_idx(b):
            pltpu.make_async_copy(src_hbm.at[pl.ds(0, CH2)], srcs[b], isems[b]).wait()
            pltpu.make_async_copy(dst_hbm.at[pl.ds(0, CH2)], ds_[b], isems[b]).wait()
            pltpu.make_async_copy(dstzl_hbm.at[cid, pl.ds(0, CH2)], dls[b],
                                  isems[b]).wait()
            pltpu.make_async_copy(erow_hbm.at[pl.ds(0, CH2)], eeis[b],
                                  isems[b]).wait()

        def fire_gather(b):
            pltpu.async_copy(kv_hbm.at[srcs[b]], krs[b], gsems[b])
            pltpu.async_copy(q_hbm.at[ds_[b]], qrs[b], gsems[b])
            pltpu.async_copy(ee_hbm.at[eeis[b]], ers[b], gsems[b])

        def wait_gather(b):
            pltpu.make_async_copy(kv_hbm.at[srcs[b]], krs[b], gsems[b]).wait()
            pltpu.make_async_copy(q_hbm.at[ds_[b]], qrs[b], gsems[b]).wait()
            pltpu.make_async_copy(ee_hbm.at[eeis[b]], ers[b], gsems[b]).wait()

        def wait_out(b):
            pltpu.make_async_copy(zvs[b], acc_z.at[dls[b]], vsems[b]).wait()

        def fire_out(b):
            pltpu.async_copy(zvs[b], acc_z.at[dls[b]], vsems[b], add=True)

        z16 = jnp.zeros((16,), jnp.float32)
        z16i = jnp.zeros((16,), jnp.int32)
        for r in range(16):
            for j in range(HID // 16):
                zrow_v[r, pl.ds(j * 16, 16)] = z16
        for b in (0, 1):
            for j in range(CH2 // 16):
                dls[b][pl.ds(j * 16, 16)] = z16i
            _zero_rows_loop(zvs[b], CH2, HID)
        _zero_shared_z(zrow_v, acc_z, sid)
        plsc.subcore_barrier()
        fire_out(0)
        fire_out(1)

        fire_idx(0, 0)
        wait_idx(0)
        fire_gather(0)
        fire_idx(1, 1)

        def pair_body(pp, _):
            for b in (0, 1):
                i = 2 * pp + b

                @pl.when(i < trips)
                def _():
                    wait_gather(b)
                    wait_out(b)

                    @pl.when(i + 1 < trips)
                    def _():
                        wait_idx(1 - b)
                        fire_gather(1 - b)

                    def gbody(g, _):
                        d16 = ds_[b][pl.ds(g * 16, 16)]
                        dlc16 = jnp.clip(d16 - lo, 0, NH - 1)
                        _edge_rows(
                            g,
                            lambda l, hh: krs[b][g * 16 + l,
                                                 pl.ds(hh * HDIM, 16)],
                            lambda l, hh: qrs[b][g * 16 + l,
                                                 pl.ds(hh * HDIM, 16)],
                            lambda l, hh: ers[b][g * 16 + l,
                                                 pl.ds(hh * HDIM, 16)]
                            * INV_SQRT_D,
                            None,
                            d16, dlc16, lo, None, zvs[b])
                        return 0

                    lax.fori_loop(0, CH2 // 16, gbody, 0)
                    fire_out(b)

                    @pl.when(i + 2 < trips)
                    def _():
                        fire_idx(b, i + 2)
            return 0

        lax.fori_loop(0, (trips + 1) // 2, pair_body, 0)
        wait_out(0)
        wait_out(1)
        plsc.subcore_barrier()
        _dump_shared_z(acc_z, zparts_hbm, cid, sid)

    return scz2


def _ln(x, g, b):
    mu = jnp.mean(x, axis=-1, keepdims=True)
    var = jnp.mean((x - mu) ** 2, axis=-1, keepdims=True)
    return (x - mu) * lax.rsqrt(var + 1e-5) * g + b


_EB = 512
_NB = 1280


def _tc_edge_body(score_ref, eidx_ref, embe_ref, woe_ref, boe_ref,
                  g1_ref, b1_ref, wf1_ref, bf1_ref, wf2_ref, bf2_ref,
                  g2_ref, b2_ref, we1_ref, out_ref):
    score = score_ref[...]
    oh = (eidx_ref[0, 0][:, None] == lax.broadcasted_iota(jnp.int32, (1, NUM_BOND), 1)
          ).astype(jnp.float32)
    e_emb = jnp.dot(oh, embe_ref[...], preferred_element_type=jnp.float32)
    x = e_emb + jnp.dot(score, woe_ref[...],
                        preferred_element_type=jnp.float32) + boe_ref[...]
    x = _ln(x, g1_ref[...], b1_ref[...])
    y = jnp.maximum(jnp.dot(x, wf1_ref[...], preferred_element_type=jnp.float32)
                    + bf1_ref[...], 0.0)
    y = jnp.dot(y, wf2_ref[...], preferred_element_type=jnp.float32) + bf2_ref[...] + x
    y = _ln(y, g2_ref[...], b2_ref[...])
    out_ref[...] = jnp.dot(y, we1_ref[...], preferred_element_type=jnp.float32)


def _tc_edge(score, eidx3d, embe, woe, boe, g1, b1, wf1, bf1, wf2, bf2, g2, b2, we1):
    nblk = N_EDGES // _EB
    wspec = lambda s: pl.BlockSpec(s, lambda i: (0,) * len(s))
    return pl.pallas_call(
        _tc_edge_body,
        grid=(nblk,),
        in_specs=[
            pl.BlockSpec((_EB, HID), lambda i: (i, 0)),
            pl.BlockSpec((1, 1, _EB), lambda i: (i, 0, 0)),
            wspec((NUM_BOND, HID)), wspec((HID, HID)), wspec((1, HID)),
            wspec((1, HID)), wspec((1, HID)),
            wspec((HID, 2 * HID)), wspec((1, 2 * HID)),
            wspec((2 * HID, HID)), wspec((1, HID)),
            wspec((1, HID)), wspec((1, HID)),
            wspec((HID, HID)),
        ],
        out_specs=pl.BlockSpec((_EB, HID), lambda i: (i, 0)),
        out_shape=jax.ShapeDtypeStruct((N_EDGES, HID), jnp.float32),
    )(score, eidx3d, embe, woe, boe, g1, b1, wf1, bf1, wf2, bf2, g2, b2, we1)


def _seg_expander():
    return (lax.broadcasted_iota(jnp.int32, (HEADS, HID), 1) // HDIM
            == lax.broadcasted_iota(jnp.int32, (HEADS, HID), 0)).astype(jnp.float32)


def _node_chain(wv, z8, h_in, woh, boh, g1, b1, wf1, bf1, wf2, bf2, g2, b2):
    zf = jnp.dot(z8, _seg_expander(), preferred_element_type=jnp.float32)
    h_att = wv / (zf + 1e-6)
    x = h_in + jnp.dot(h_att, woh, preferred_element_type=jnp.float32) + boh
    x = _ln(x, g1, b1)
    y = jnp.maximum(jnp.dot(x, wf1, preferred_element_type=jnp.float32) + bf1, 0.0)
    y = jnp.dot(y, wf2, preferred_element_type=jnp.float32) + bf2 + x
    return _ln(y, g2, b2)


def _tc_h1_body(wv_ref, z_ref, hidx_ref, embh_ref, woh_ref, boh_ref,
                g1_ref, b1_ref, wf1_ref, bf1_ref, wf2_ref, bf2_ref,
                g2_ref, b2_ref, wk_ref, wq_ref, wv2_ref,
                hf_ref, kv_ref, q_ref):
    oh = (hidx_ref[...] == lax.broadcasted_iota(jnp.int32, (1, NUM_ATOM), 1)
          ).astype(jnp.float32)
    h0 = jnp.dot(oh, embh_ref[...], preferred_element_type=jnp.float32)
    hf = _node_chain(wv_ref[...], z_ref[...], h0,
                     woh_ref[...], boh_ref[...],
                     g1_ref[...], b1_ref[...], wf1_ref[...], bf1_ref[...],
                     wf2_ref[...], bf2_ref[...], g2_ref[...], b2_ref[...])
    hf_ref[...] = hf
    kv_ref[:, :HID] = jnp.dot(hf, wk_ref[...], preferred_element_type=jnp.float32)
    kv_ref[:, HID:] = jnp.dot(hf, wv2_ref[...], preferred_element_type=jnp.float32)
    q_ref[...] = jnp.dot(hf, wq_ref[...], preferred_element_type=jnp.float32)


def _tc_h1(wv, z8, hidx2d, embh, woh, boh, g1, b1, wf1, bf1, wf2, bf2,
           g2, b2, wk, wq, wvp):
    nblk = NPAD // _NB
    wspec = lambda s: pl.BlockSpec(s, lambda i: (0,) * len(s))
    outs = [jax.ShapeDtypeStruct((NPAD, HID), jnp.float32),
            jax.ShapeDtypeStruct((NPAD, 2 * HID), jnp.float32),
            jax.ShapeDtypeStruct((NPAD, HID), jnp.float32)]
    return pl.pallas_call(
        _tc_h1_body,
        grid=(nblk,),
        in_specs=[
            pl.BlockSpec((_NB, HID), lambda i: (i, 0)),
            pl.BlockSpec((_NB, HEADS), lambda i: (i, 0)),
            pl.BlockSpec((_NB, 1), lambda i: (i, 0)),
            wspec((NUM_ATOM, HID)), wspec((HID, HID)), wspec((1, HID)),
            wspec((1, HID)), wspec((1, HID)),
            wspec((HID, 2 * HID)), wspec((1, 2 * HID)),
            wspec((2 * HID, HID)), wspec((1, HID)),
            wspec((1, HID)), wspec((1, HID)),
            wspec((HID, HID)), wspec((HID, HID)), wspec((HID, HID)),
        ],
        out_specs=[pl.BlockSpec((_NB, HID), lambda i: (i, 0)),
                   pl.BlockSpec((_NB, 2 * HID), lambda i: (i, 0)),
                   pl.BlockSpec((_NB, HID), lambda i: (i, 0))],
        out_shape=outs,
    )(wv, z8, hidx2d, embh, woh, boh, g1, b1, wf1, bf1, wf2, bf2,
      g2, b2, wk, wq, wvp)


def _tc_final_body(wv_ref, z_ref, hf_ref, woh_ref, boh_ref,
                   g1_ref, b1_ref, wf1_ref, bf1_ref, wf2_ref, bf2_ref,
                   g2_ref, b2_ref, out_ref):
    i = pl.program_id(0)
    h2 = _node_chain(wv_ref[...], z_ref[...], hf_ref[...],
                     woh_ref[...], boh_ref[...],
                     g1_ref[...], b1_ref[...], wf1_ref[...], bf1_ref[...],
                     wf2_ref[...], bf2_ref[...], g2_ref[...], b2_ref[...])
    rows = lax.broadcasted_iota(jnp.int32, (_NB, 1), 0) + i * _NB
    mask = (rows < N_NODES).astype(jnp.float32)
    part = jnp.sum(h2 * mask, axis=0, keepdims=True) * (1.0 / N_NODES)

    @pl.when(i == 0)
    def _():
        out_ref[...] = jnp.zeros_like(out_ref)

    out_ref[...] += part


def _tc_final(wv, z8, hf, woh, boh, g1, b1, wf1, bf1, wf2, bf2, g2, b2):
    nblk = NPAD // _NB
    wspec = lambda s: pl.BlockSpec(s, lambda i: (0,) * len(s))
    return pl.pallas_call(
        _tc_final_body,
        grid=(nblk,),
        in_specs=[
            pl.BlockSpec((_NB, HID), lambda i: (i, 0)),
            pl.BlockSpec((_NB, HEADS), lambda i: (i, 0)),
            pl.BlockSpec((_NB, HID), lambda i: (i, 0)),
            wspec((HID, HID)), wspec((1, HID)),
            wspec((1, HID)), wspec((1, HID)),
            wspec((HID, 2 * HID)), wspec((1, 2 * HID)),
            wspec((2 * HID, HID)), wspec((1, HID)),
            wspec((1, HID)), wspec((1, HID)),
        ],
        out_specs=pl.BlockSpec((1, HID), lambda i: (0, 0)),
        out_shape=jax.ShapeDtypeStruct((1, HID), jnp.float32),
    )(wv, z8, hf, woh, boh, g1, b1, wf1, bf1, wf2, bf2, g2, b2)


def _zsum(zparts):
    return zparts.reshape(NPAD, HEADS)


def kernel(params, h, e, edge_index):
    p = params
    src = edge_index[0].astype(jnp.int32)
    dst = edge_index[1].astype(jnp.int32)
    hidx = h.astype(jnp.int32)
    eidx = e.astype(jnp.int32)

    mk = p['emb_h'] @ p['Wk'][0]
    mq = p['emb_h'] @ p['Wq'][0]
    mv = p['emb_h'] @ p['Wv'][0]
    me = (p['emb_e'] @ p['We'][0]) * INV_SQRT_D

    aidx = hidx[src]
    bidx = hidx[dst]
    dstl = jnp.stack([jnp.clip(dst, 0, NH - 1),
                      jnp.clip(dst - NH, 0, NH - 1)]).astype(jnp.int32)
    dstzl = dstl >> 4
    erow = jnp.arange(N_EDGES, dtype=jnp.int32)

    sc1 = _make_sc1()
    score1, parts1 = sc1(mk, mq, mv, me, aidx, bidx, eidx,
                         dst, dstl, erow)
    wv1 = parts1.reshape(NPAD, HID)
    scz1 = _make_scz1()
    z1 = _zsum(scz1(mk, mq, me, aidx, bidx, eidx, dst, dstzl))

    eidx3d = eidx.reshape(N_EDGES // _EB, 1, _EB)
    ee2 = _tc_edge(score1, eidx3d, p['emb_e'],
                   p['WOe'][0], p['bOe'][0][None], p['ln1e_g'][0][None],
                   p['ln1e_b'][0][None], p['Wf1e'][0], p['bf1e'][0][None],
                   p['Wf2e'][0], p['bf2e'][0][None], p['ln2e_g'][0][None],
                   p['ln2e_b'][0][None], p['We'][1])

    hidx2d = jnp.pad(hidx, (0, NPAD - N_NODES)).reshape(NPAD, 1)
    hf, kv2, q2 = _tc_h1(wv1, z1, hidx2d, p['emb_h'],
                         p['WOh'][0], p['bOh'][0][None], p['ln1h_g'][0][None],
                         p['ln1h_b'][0][None], p['Wf1h'][0], p['bf1h'][0][None],
                         p['Wf2h'][0], p['bf2h'][0][None], p['ln2h_g'][0][None],
                         p['ln2h_b'][0][None], p['Wk'][1], p['Wq'][1],
                         p['Wv'][1])

    sc2 = _make_sc2()
    parts2 = sc2(kv2, q2, ee2, src, dst, dstl, erow)
    wv2 = parts2.reshape(NPAD, HID)
    scz2 = _make_scz2()
    z2 = _zsum(scz2(kv2, q2, ee2, src, dst, dstzl, erow))

    hg = _tc_final(wv2, z2, hf,
                   p['WOh'][1], p['bOh'][1][None], p['ln1h_g'][1][None],
                   p['ln1h_b'][1][None], p['Wf1h'][1], p['bf1h'][1][None],
                   p['Wf2h'][1], p['bf2h'][1][None], p['ln2h_g'][1][None],
                   p['ln2h_b'][1][None])
    return hg[0]
